# baseline (speedup 1.0000x reference)
import functools

import jax
import jax.numpy as jnp
from jax import lax
from jax.experimental import pallas as pl
from jax.experimental.pallas import tpu as pltpu
from jax.experimental.pallas import tpu_sc as plsc

_B = 16384
_D = 128
_K = 100
_ALPHA = 0.5
_EPS = 1e-6

_INFO = plsc.get_sparse_core_info()
_NC = _INFO.num_cores
_NS = _INFO.num_subcores
_NW = _NC * _NS
_RW = _B // _NW
_CH = 128
_NCHUNK = _RW // _CH
_SW = 144

_mesh = plsc.VectorSubcoreMesh(core_axis_name="c", subcore_axis_name="s")


@functools.partial(
    pl.kernel,
    mesh=_mesh,
    out_type=(jax.ShapeDtypeStruct((_NW, 16), jnp.float32),
              jax.ShapeDtypeStruct((_NW, _K, _SW), jnp.float32)),
    compiler_params=pltpu.CompilerParams(needs_layout_passes=False),
    scratch_types=[
        pltpu.VMEM((_RW,), jnp.int32),
        pltpu.VMEM((_K, _SW), jnp.float32),
        pltpu.VMEM((_CH, _D), jnp.float32),
        pltpu.VMEM((_CH, _D), jnp.float32),
        pltpu.VMEM((16,), jnp.float32),
        pltpu.SemaphoreType.DMA,
        pltpu.SemaphoreType.DMA,
    ],
)
def _center_partials(emb_hbm, lab_hbm, cen_hbm, acc_hbm, seg_hbm,
                     lab_v, seg_v, emb0, emb1, acc_v, sem0, sem1):
    del cen_hbm
    wid = lax.axis_index("s") * _NC + lax.axis_index("c")
    base = wid * _RW
    pltpu.sync_copy(lab_hbm.at[pl.ds(base, _RW)], lab_v)

    bufs = (emb0, emb1)
    sems = (sem0, sem1)
    copies = [None, None]
    copies[0] = pltpu.async_copy(emb_hbm.at[pl.ds(base, _CH)], emb0, sem0)

    lane = lax.iota(jnp.int32, 16)
    zeros16 = jnp.zeros((16,), jnp.float32)

    def zbody(r, carry):
        for g in range(_SW // 16):
            seg_v[r, pl.ds(g * 16, 16)] = zeros16
        return carry

    lax.fori_loop(0, _K, zbody, 0)

    cols = [lane + g * 16 for g in range(_D // 16)]
    cnt_col = lane + _D
    cnt_val = jnp.where(lane == 0, 1.0, 0.0)
    acc = jnp.zeros((16,), jnp.float32)
    for chunk in range(_NCHUNK):
        cur = chunk % 2
        nxt = 1 - cur
        if chunk + 1 < _NCHUNK:
            copies[nxt] = pltpu.async_copy(
                emb_hbm.at[pl.ds(base + (chunk + 1) * _CH, _CH)],
                bufs[nxt], sems[nxt])
        copies[cur].wait()
        ebuf = bufs[cur]
        row0 = chunk * _CH

        def body(r, a, ebuf=ebuf, row0=row0):
            lbl = plsc.load_gather(lab_v, [jnp.full((16,), row0 + r, jnp.int32)])
            for g in range(_D // 16):
                e = ebuf[r, pl.ds(g * 16, 16)]
                a = a + e * e
                plsc.addupdate_scatter(seg_v, [lbl, cols[g]], e)
            plsc.addupdate_scatter(seg_v, [lbl, cnt_col], cnt_val)
            return a

        acc = lax.fori_loop(0, _CH, body, acc)
    acc_v[...] = acc
    pltpu.sync_copy(acc_v, acc_hbm.at[wid])
    pltpu.sync_copy(seg_v, seg_hbm.at[wid])


def _sep_kernel(cen_ref, out_ref):
    c = cen_ref[...]
    g = lax.dot_general(c, c, (((1,), (1,)), ((), ())),
                        precision=lax.Precision.HIGHEST)
    row = lax.broadcasted_iota(jnp.int32, (_K, _K), 0)
    col = lax.broadcasted_iota(jnp.int32, (_K, _K), 1)
    eye = jnp.where(row == col, 1.0, 0.0)
    n_col = jnp.sum(g * eye, axis=1, keepdims=True)
    n_row = jnp.sum(g * eye, axis=0, keepdims=True)
    s_col = jnp.sum(c, axis=1, keepdims=True)
    s_row = jnp.sum(eye * s_col, axis=0, keepdims=True)
    d2 = (n_col + n_row - 2.0 * g
          + (2.0 * _EPS) * (s_col - s_row) + _D * _EPS * _EPS)
    dist = jnp.sqrt(jnp.maximum(d2, 0.0)) * (1.0 - eye)
    out_ref[...] = jnp.reshape(jnp.sum(dist), (1, 1))


def _combine_kernel(acc_ref, seg_ref, cen_ref, sep_ref, out_ref):
    seg = jnp.sum(seg_ref[...], axis=0)
    c = cen_ref[...]
    s_dot_c = jnp.sum(seg[:, :_D] * c)
    counts = seg[:, _D:_D + 1]
    norms = jnp.sum(c * c, axis=1, keepdims=True)
    e_sq = jnp.sum(acc_ref[...])
    center_sum = e_sq - 2.0 * s_dot_c + jnp.sum(counts * norms)
    total = (center_sum / (_B * _D)
             - _ALPHA * sep_ref[0, 0] / (_K * (_K - 1)))
    out_ref[...] = jnp.reshape(total, (1, 1))


def kernel(embeddings, cluster_labels, centers):
    sep = pl.pallas_call(
        _sep_kernel,
        out_shape=jax.ShapeDtypeStruct((1, 1), jnp.float32),
    )(centers)
    acc, seg = _center_partials(embeddings, cluster_labels, centers)
    total = pl.pallas_call(
        _combine_kernel,
        out_shape=jax.ShapeDtypeStruct((1, 1), jnp.float32),
    )(acc, seg, centers, sep)
    return total.reshape(())


# speedup vs baseline: 1.3440x; 1.3440x over previous
import functools

import jax
import jax.numpy as jnp
from jax import lax
from jax.experimental import pallas as pl
from jax.experimental.pallas import tpu as pltpu
from jax.experimental.pallas import tpu_sc as plsc

_B = 16384
_D = 128
_K = 100
_ALPHA = 0.5
_EPS = 1e-6

_INFO = plsc.get_sparse_core_info()
_NC = _INFO.num_cores
_NS = _INFO.num_subcores
_NW = _NC * _NS
_RW = _B // _NW
_CH = 128
_NCHUNK = _RW // _CH

_mesh = plsc.VectorSubcoreMesh(core_axis_name="c", subcore_axis_name="s")


@functools.partial(
    pl.kernel,
    mesh=_mesh,
    out_type=jax.ShapeDtypeStruct((_NW, 16), jnp.float32),
    compiler_params=pltpu.CompilerParams(needs_layout_passes=False),
    scratch_types=[
        pltpu.VMEM((_RW,), jnp.int32),
        pltpu.VMEM((_K, _D), jnp.float32),
        pltpu.VMEM((_CH, _D), jnp.float32),
        pltpu.VMEM((_CH, _D), jnp.float32),
        pltpu.VMEM((16,), jnp.float32),
        pltpu.SemaphoreType.DMA,
        pltpu.SemaphoreType.DMA,
    ],
)
def _center_partials(emb_hbm, lab_hbm, cen_hbm, out_hbm,
                     lab_v, tab_v, emb0, emb1, acc_v, sem0, sem1):
    wid = lax.axis_index("s") * _NC + lax.axis_index("c")
    base = wid * _RW
    pltpu.sync_copy(cen_hbm, tab_v)
    pltpu.sync_copy(lab_hbm.at[pl.ds(base, _RW)], lab_v)

    bufs = (emb0, emb1)
    sems = (sem0, sem1)
    copies = [None, None]
    copies[0] = pltpu.async_copy(emb_hbm.at[pl.ds(base, _CH)], emb0, sem0)

    cols = [lax.iota(jnp.int32, 16) + g * 16 for g in range(_D // 16)]
    acc = jnp.zeros((16,), jnp.float32)
    for chunk in range(_NCHUNK):
        cur = chunk % 2
        nxt = 1 - cur
        if chunk + 1 < _NCHUNK:
            copies[nxt] = pltpu.async_copy(
                emb_hbm.at[pl.ds(base + (chunk + 1) * _CH, _CH)],
                bufs[nxt], sems[nxt])
        copies[cur].wait()
        ebuf = bufs[cur]
        row0 = chunk * _CH

        @plsc.parallel_loop(0, _CH, unroll=2, carry=acc)
        def body(r, a, ebuf=ebuf, row0=row0):
            lbl = plsc.load_gather(lab_v, [jnp.full((16,), row0 + r, jnp.int32)])
            for g in range(_D // 16):
                c = plsc.load_gather(tab_v, [lbl, cols[g]])
                e = ebuf[r, pl.ds(g * 16, 16)]
                d = e - c
                a = a + d * d
            return a

        acc = body
    acc_v[...] = acc
    pltpu.sync_copy(acc_v, out_hbm.at[wid])


def _sep_kernel(cen_ref, out_ref):
    c = cen_ref[...]
    g = lax.dot_general(c, c, (((1,), (1,)), ((), ())),
                        precision=lax.Precision.HIGHEST)
    row = lax.broadcasted_iota(jnp.int32, (_K, _K), 0)
    col = lax.broadcasted_iota(jnp.int32, (_K, _K), 1)
    eye = jnp.where(row == col, 1.0, 0.0)
    n_col = jnp.sum(g * eye, axis=1, keepdims=True)
    n_row = jnp.sum(g * eye, axis=0, keepdims=True)
    s_col = jnp.sum(c, axis=1, keepdims=True)
    s_row = jnp.sum(eye * s_col, axis=0, keepdims=True)
    d2 = (n_col + n_row - 2.0 * g
          + (2.0 * _EPS) * (s_col - s_row) + _D * _EPS * _EPS)
    dist = jnp.sqrt(jnp.maximum(d2, 0.0)) * (1.0 - eye)
    out_ref[...] = jnp.reshape(jnp.sum(dist), (1, 1))


def kernel(embeddings, cluster_labels, centers):
    partials = _center_partials(embeddings, cluster_labels, centers)
    sep = pl.pallas_call(
        _sep_kernel,
        out_shape=jax.ShapeDtypeStruct((1, 1), jnp.float32),
    )(centers)
    total = jnp.sum(partials) / (_B * _D) - _ALPHA * sep[0, 0] / (_K * (_K - 1))
    return total


# speedup vs baseline: 1.4161x; 1.0537x over previous
import functools

import jax
import jax.numpy as jnp
from jax import lax
from jax.experimental import pallas as pl
from jax.experimental.pallas import tpu as pltpu
from jax.experimental.pallas import tpu_sc as plsc

_B = 16384
_D = 128
_K = 100
_ALPHA = 0.5
_EPS = 1e-6

_SB = 8192
_TCB = 1024
_NB = (_B - _SB) // _TCB

_INFO = plsc.get_sparse_core_info()
_NC = _INFO.num_cores
_NS = _INFO.num_subcores
_NW = _NC * _NS
_RW = _SB // _NW
_CH = 128
_NCHUNK = _RW // _CH

_mesh = plsc.VectorSubcoreMesh(core_axis_name="c", subcore_axis_name="s")


@functools.partial(
    pl.kernel,
    mesh=_mesh,
    out_type=jax.ShapeDtypeStruct((_NW, 16), jnp.float32),
    compiler_params=pltpu.CompilerParams(needs_layout_passes=False),
    scratch_types=[
        pltpu.VMEM((_RW,), jnp.int32),
        pltpu.VMEM((_K, _D), jnp.float32),
        pltpu.VMEM((_CH, _D), jnp.float32),
        pltpu.VMEM((_CH, _D), jnp.float32),
        pltpu.VMEM((16,), jnp.float32),
        pltpu.SemaphoreType.DMA,
        pltpu.SemaphoreType.DMA,
    ],
)
def _center_partials(emb_hbm, lab_hbm, cen_hbm, out_hbm,
                     lab_v, tab_v, emb0, emb1, acc_v, sem0, sem1):
    wid = lax.axis_index("s") * _NC + lax.axis_index("c")
    base = wid * _RW
    pltpu.sync_copy(cen_hbm, tab_v)
    pltpu.sync_copy(lab_hbm.at[pl.ds(base, _RW)], lab_v)

    bufs = (emb0, emb1)
    sems = (sem0, sem1)
    copies = [None, None]
    copies[0] = pltpu.async_copy(emb_hbm.at[pl.ds(base, _CH)], emb0, sem0)

    cols = [lax.iota(jnp.int32, 16) + g * 16 for g in range(_D // 16)]
    acc = jnp.zeros((16,), jnp.float32)
    for chunk in range(_NCHUNK):
        cur = chunk % 2
        nxt = 1 - cur
        if chunk + 1 < _NCHUNK:
            copies[nxt] = pltpu.async_copy(
                emb_hbm.at[pl.ds(base + (chunk + 1) * _CH, _CH)],
                bufs[nxt], sems[nxt])
        copies[cur].wait()
        ebuf = bufs[cur]
        row0 = chunk * _CH

        @plsc.parallel_loop(0, _CH, unroll=2, carry=acc)
        def body(r, a, ebuf=ebuf, row0=row0):
            lbl = plsc.load_gather(lab_v, [jnp.full((16,), row0 + r, jnp.int32)])
            for g in range(_D // 16):
                c = plsc.load_gather(tab_v, [lbl, cols[g]])
                e = ebuf[r, pl.ds(g * 16, 16)]
                d = e - c
                a = a + d * d
            return a

        acc = body
    acc_v[...] = acc
    pltpu.sync_copy(acc_v, out_hbm.at[wid])


def _tc_mse_kernel(emb_ref, lab_ref, cen_ref, out_ref, accm_ref, accc_ref, e2_ref):
    i = pl.program_id(0)

    @pl.when(i == 0)
    def _():
        accm_ref[...] = jnp.zeros_like(accm_ref)
        accc_ref[...] = jnp.zeros_like(accc_ref)
        e2_ref[0] = 0.0

    e = emb_ref[...]
    lab = lab_ref[0]
    kk = lax.broadcasted_iota(jnp.int32, (_K, _TCB), 0)
    onehot_t = jnp.where(lab == kk, 1.0, 0.0)
    m = lax.dot_general(onehot_t, e, (((1,), (0,)), ((), ())),
                        precision=lax.Precision.HIGHEST)
    accm_ref[...] += m
    accc_ref[...] += jnp.sum(onehot_t, axis=1, keepdims=True)
    e2_ref[0] += jnp.sum(e * e)

    @pl.when(i == _NB - 1)
    def _():
        c = cen_ref[...]
        norms = jnp.sum(c * c, axis=1, keepdims=True)
        center_sum = (e2_ref[0]
                      - 2.0 * jnp.sum(accm_ref[...] * c)
                      + jnp.sum(accc_ref[...] * norms))
        out_ref[...] = jnp.reshape(center_sum, (1, 1))


def _sep_kernel(cen_ref, out_ref):
    c = cen_ref[...]
    g = lax.dot_general(c, c, (((1,), (1,)), ((), ())),
                        precision=lax.Precision.HIGHEST)
    row = lax.broadcasted_iota(jnp.int32, (_K, _K), 0)
    col = lax.broadcasted_iota(jnp.int32, (_K, _K), 1)
    eye = jnp.where(row == col, 1.0, 0.0)
    n_col = jnp.sum(g * eye, axis=1, keepdims=True)
    n_row = jnp.sum(g * eye, axis=0, keepdims=True)
    s_col = jnp.sum(c, axis=1, keepdims=True)
    s_row = jnp.sum(eye * s_col, axis=0, keepdims=True)
    d2 = (n_col + n_row - 2.0 * g
          + (2.0 * _EPS) * (s_col - s_row) + _D * _EPS * _EPS)
    dist = jnp.sqrt(jnp.maximum(d2, 0.0)) * (1.0 - eye)
    out_ref[...] = jnp.reshape(jnp.sum(dist), (1, 1))


def kernel(embeddings, cluster_labels, centers):
    partials = _center_partials(embeddings, cluster_labels, centers)
    lab3d = cluster_labels.reshape(_B // _TCB, 1, _TCB)
    tc_center = pl.pallas_call(
        _tc_mse_kernel,
        grid=(_NB,),
        in_specs=[
            pl.BlockSpec((_TCB, _D), lambda i: (i + _SB // _TCB, 0)),
            pl.BlockSpec((1, 1, _TCB), lambda i: (i + _SB // _TCB, 0, 0)),
            pl.BlockSpec((_K, _D), lambda i: (0, 0)),
        ],
        out_specs=pl.BlockSpec((1, 1), lambda i: (0, 0)),
        out_shape=jax.ShapeDtypeStruct((1, 1), jnp.float32),
        scratch_shapes=[
            pltpu.VMEM((_K, _D), jnp.float32),
            pltpu.VMEM((_K, 1), jnp.float32),
            pltpu.SMEM((1,), jnp.float32),
        ],
    )(embeddings, lab3d, centers)
    sep = pl.pallas_call(
        _sep_kernel,
        out_shape=jax.ShapeDtypeStruct((1, 1), jnp.float32),
    )(centers)
    center_sum = jnp.sum(partials) + tc_center[0, 0]
    total = center_sum / (_B * _D) - _ALPHA * sep[0, 0] / (_K * (_K - 1))
    return total


# speedup vs baseline: 1.4290x; 1.0091x over previous
import functools

import jax
import jax.numpy as jnp
from jax import lax
from jax.experimental import pallas as pl
from jax.experimental.pallas import tpu as pltpu
from jax.experimental.pallas import tpu_sc as plsc

_B = 16384
_D = 128
_K = 100
_ALPHA = 0.5
_EPS = 1e-6

_SB = 6144
_TCB = 1024
_NB = (_B - _SB) // _TCB

_INFO = plsc.get_sparse_core_info()
_NC = _INFO.num_cores
_NS = _INFO.num_subcores
_NW = _NC * _NS
_RW = _SB // _NW
_CH = 96
_NCHUNK = _RW // _CH

_mesh = plsc.VectorSubcoreMesh(core_axis_name="c", subcore_axis_name="s")


@functools.partial(
    pl.kernel,
    mesh=_mesh,
    out_type=jax.ShapeDtypeStruct((_NW, 16), jnp.float32),
    compiler_params=pltpu.CompilerParams(needs_layout_passes=False),
    scratch_types=[
        pltpu.VMEM((_RW,), jnp.int32),
        pltpu.VMEM((_K, _D), jnp.float32),
        pltpu.VMEM((_CH, _D), jnp.float32),
        pltpu.VMEM((_CH, _D), jnp.float32),
        pltpu.VMEM((16,), jnp.float32),
        pltpu.SemaphoreType.DMA,
        pltpu.SemaphoreType.DMA,
    ],
)
def _center_partials(emb_hbm, lab_hbm, cen_hbm, out_hbm,
                     lab_v, tab_v, emb0, emb1, acc_v, sem0, sem1):
    wid = lax.axis_index("s") * _NC + lax.axis_index("c")
    base = wid * _RW
    pltpu.sync_copy(cen_hbm, tab_v)
    pltpu.sync_copy(lab_hbm.at[pl.ds(base, _RW)], lab_v)

    bufs = (emb0, emb1)
    sems = (sem0, sem1)
    copies = [None, None]
    copies[0] = pltpu.async_copy(emb_hbm.at[pl.ds(base, _CH)], emb0, sem0)

    cols = [lax.iota(jnp.int32, 16) + g * 16 for g in range(_D // 16)]
    acc = jnp.zeros((16,), jnp.float32)
    for chunk in range(_NCHUNK):
        cur = chunk % 2
        nxt = 1 - cur
        if chunk + 1 < _NCHUNK:
            copies[nxt] = pltpu.async_copy(
                emb_hbm.at[pl.ds(base + (chunk + 1) * _CH, _CH)],
                bufs[nxt], sems[nxt])
        copies[cur].wait()
        ebuf = bufs[cur]
        row0 = chunk * _CH

        @plsc.parallel_loop(0, _CH, unroll=2, carry=acc)
        def body(r, a, ebuf=ebuf, row0=row0):
            lbl = plsc.load_gather(lab_v, [jnp.full((16,), row0 + r, jnp.int32)])
            for g in range(_D // 16):
                c = plsc.load_gather(tab_v, [lbl, cols[g]])
                e = ebuf[r, pl.ds(g * 16, 16)]
                d = e - c
                a = a + d * d
            return a

        acc = body
    acc_v[...] = acc
    pltpu.sync_copy(acc_v, out_hbm.at[wid])


def _tc_mse_kernel(emb_ref, lab_ref, cen_ref, out_ref, accm_ref, accc_ref, e2_ref):
    i = pl.program_id(0)

    @pl.when(i == 0)
    def _():
        accm_ref[...] = jnp.zeros_like(accm_ref)
        accc_ref[...] = jnp.zeros_like(accc_ref)
        e2_ref[0] = 0.0

    e = emb_ref[...]
    lab = lab_ref[0]
    kk = lax.broadcasted_iota(jnp.int32, (_K, _TCB), 0)
    onehot_t = jnp.where(lab == kk, 1.0, 0.0)
    m = lax.dot_general(onehot_t.astype(jnp.bfloat16), e.astype(jnp.bfloat16),
                        (((1,), (0,)), ((), ())),
                        preferred_element_type=jnp.float32)
    accm_ref[...] += m
    accc_ref[...] += jnp.sum(onehot_t, axis=1, keepdims=True)
    e2_ref[0] += jnp.sum(e * e)

    @pl.when(i == _NB - 1)
    def _():
        c = cen_ref[...]
        norms = jnp.sum(c * c, axis=1, keepdims=True)
        center_sum = (e2_ref[0]
                      - 2.0 * jnp.sum(accm_ref[...] * c)
                      + jnp.sum(accc_ref[...] * norms))
        out_ref[...] = jnp.reshape(center_sum, (1, 1))


def _sep_kernel(cen_ref, out_ref):
    c = cen_ref[...]
    g = lax.dot_general(c, c, (((1,), (1,)), ((), ())),
                        precision=lax.Precision.HIGHEST)
    row = lax.broadcasted_iota(jnp.int32, (_K, _K), 0)
    col = lax.broadcasted_iota(jnp.int32, (_K, _K), 1)
    eye = jnp.where(row == col, 1.0, 0.0)
    n_col = jnp.sum(g * eye, axis=1, keepdims=True)
    n_row = jnp.sum(g * eye, axis=0, keepdims=True)
    s_col = jnp.sum(c, axis=1, keepdims=True)
    s_row = jnp.sum(eye * s_col, axis=0, keepdims=True)
    d2 = (n_col + n_row - 2.0 * g
          + (2.0 * _EPS) * (s_col - s_row) + _D * _EPS * _EPS)
    dist = jnp.sqrt(jnp.maximum(d2, 0.0)) * (1.0 - eye)
    out_ref[...] = jnp.reshape(jnp.sum(dist), (1, 1))


def kernel(embeddings, cluster_labels, centers):
    partials = _center_partials(embeddings, cluster_labels, centers)
    lab3d = cluster_labels.reshape(_B // _TCB, 1, _TCB)
    tc_center = pl.pallas_call(
        _tc_mse_kernel,
        grid=(_NB,),
        in_specs=[
            pl.BlockSpec((_TCB, _D), lambda i: (i + _SB // _TCB, 0)),
            pl.BlockSpec((1, 1, _TCB), lambda i: (i + _SB // _TCB, 0, 0)),
            pl.BlockSpec((_K, _D), lambda i: (0, 0)),
        ],
        out_specs=pl.BlockSpec((1, 1), lambda i: (0, 0)),
        out_shape=jax.ShapeDtypeStruct((1, 1), jnp.float32),
        scratch_shapes=[
            pltpu.VMEM((_K, _D), jnp.float32),
            pltpu.VMEM((_K, 1), jnp.float32),
            pltpu.SMEM((1,), jnp.float32),
        ],
    )(embeddings, lab3d, centers)
    sep = pl.pallas_call(
        _sep_kernel,
        out_shape=jax.ShapeDtypeStruct((1, 1), jnp.float32),
    )(centers)
    center_sum = jnp.sum(partials) + tc_center[0, 0]
    total = center_sum / (_B * _D) - _ALPHA * sep[0, 0] / (_K * (_K - 1))
    return total


# speedup vs baseline: 1.4932x; 1.0449x over previous
import functools

import jax
import jax.numpy as jnp
from jax import lax
from jax.experimental import pallas as pl
from jax.experimental.pallas import tpu as pltpu
from jax.experimental.pallas import tpu_sc as plsc

_B = 16384
_D = 128
_K = 100
_ALPHA = 0.5
_EPS = 1e-6

_SB = 6144
_TCB = 1024
_NB = (_B - _SB) // _TCB

_INFO = plsc.get_sparse_core_info()
_NC = _INFO.num_cores
_NS = _INFO.num_subcores
_NW = _NC * _NS
_RW = _SB // _NW
_CH = 96
_NCHUNK = _RW // _CH

_mesh = plsc.VectorSubcoreMesh(core_axis_name="c", subcore_axis_name="s")


@functools.partial(
    pl.kernel,
    mesh=_mesh,
    out_type=jax.ShapeDtypeStruct((_NW, 16), jnp.float32),
    compiler_params=pltpu.CompilerParams(needs_layout_passes=False),
    scratch_types=[
        pltpu.VMEM((_RW,), jnp.int32),
        pltpu.VMEM((_K, _D), jnp.float32),
        pltpu.VMEM_SHARED((_K, _D), jnp.float32),
        pltpu.VMEM((_CH, _D), jnp.float32),
        pltpu.VMEM((_CH, _D), jnp.float32),
        pltpu.VMEM((16,), jnp.float32),
        pltpu.SemaphoreType.DMA,
        pltpu.SemaphoreType.DMA,
    ],
)
def _center_partials(emb_hbm, lab_hbm, cen_hbm, out_hbm,
                     lab_v, tab_v, tab_sp, emb0, emb1, acc_v, sem0, sem1):
    wid = lax.axis_index("s") * _NC + lax.axis_index("c")
    base = wid * _RW

    bufs = (emb0, emb1)
    sems = (sem0, sem1)
    copies = [None, None]
    copies[0] = pltpu.async_copy(emb_hbm.at[pl.ds(base, _CH)], emb0, sem0)
    if _NCHUNK > 1:
        copies[1] = pltpu.async_copy(
            emb_hbm.at[pl.ds(base + _CH, _CH)], emb1, sem1)

    @pl.when(lax.axis_index("s") == 0)
    def _():
        pltpu.sync_copy(cen_hbm, tab_sp)

    plsc.subcore_barrier()
    pltpu.sync_copy(tab_sp, tab_v)
    pltpu.sync_copy(lab_hbm.at[pl.ds(base, _RW)], lab_v)

    cols = [lax.iota(jnp.int32, 16) + g * 16 for g in range(_D // 16)]
    acc = jnp.zeros((16,), jnp.float32)
    for chunk in range(_NCHUNK):
        cur = chunk % 2
        copies[cur].wait()
        ebuf = bufs[cur]
        row0 = chunk * _CH

        @plsc.parallel_loop(0, _CH, unroll=2, carry=acc)
        def body(r, a, ebuf=ebuf, row0=row0):
            lbl = plsc.load_gather(lab_v, [jnp.full((16,), row0 + r, jnp.int32)])
            for g in range(_D // 16):
                c = plsc.load_gather(tab_v, [lbl, cols[g]])
                e = ebuf[r, pl.ds(g * 16, 16)]
                d = e - c
                a = a + d * d
            return a

        acc = body
    acc_v[...] = acc
    pltpu.sync_copy(acc_v, out_hbm.at[wid])


def _tc_mse_kernel(emb_ref, lab_ref, cen_ref, out_ref, accm_ref, accc_ref, e2_ref):
    i = pl.program_id(0)

    @pl.when(i == 0)
    def _():
        accm_ref[...] = jnp.zeros_like(accm_ref)
        accc_ref[...] = jnp.zeros_like(accc_ref)
        e2_ref[0] = 0.0

    e = emb_ref[...]
    lab = lab_ref[0]
    kk = lax.broadcasted_iota(jnp.int32, (_K, _TCB), 0)
    onehot_t = jnp.where(lab == kk, 1.0, 0.0)
    m = lax.dot_general(onehot_t.astype(jnp.bfloat16), e.astype(jnp.bfloat16),
                        (((1,), (0,)), ((), ())),
                        preferred_element_type=jnp.float32)
    accm_ref[...] += m
    accc_ref[...] += jnp.sum(onehot_t, axis=1, keepdims=True)
    e2_ref[0] += jnp.sum(e * e)

    @pl.when(i == _NB - 1)
    def _():
        c = cen_ref[...]
        norms = jnp.sum(c * c, axis=1, keepdims=True)
        center_sum = (e2_ref[0]
                      - 2.0 * jnp.sum(accm_ref[...] * c)
                      + jnp.sum(accc_ref[...] * norms))
        out_ref[...] = jnp.reshape(center_sum, (1, 1))


def _sep_kernel(cen_ref, out_ref):
    c = cen_ref[...]
    g = lax.dot_general(c, c, (((1,), (1,)), ((), ())),
                        precision=lax.Precision.HIGHEST)
    row = lax.broadcasted_iota(jnp.int32, (_K, _K), 0)
    col = lax.broadcasted_iota(jnp.int32, (_K, _K), 1)
    eye = jnp.where(row == col, 1.0, 0.0)
    n_col = jnp.sum(g * eye, axis=1, keepdims=True)
    n_row = jnp.sum(g * eye, axis=0, keepdims=True)
    s_col = jnp.sum(c, axis=1, keepdims=True)
    s_row = jnp.sum(eye * s_col, axis=0, keepdims=True)
    d2 = (n_col + n_row - 2.0 * g
          + (2.0 * _EPS) * (s_col - s_row) + _D * _EPS * _EPS)
    dist = jnp.sqrt(jnp.maximum(d2, 0.0)) * (1.0 - eye)
    out_ref[...] = jnp.reshape(jnp.sum(dist), (1, 1))


def kernel(embeddings, cluster_labels, centers):
    partials = _center_partials(embeddings, cluster_labels, centers)
    lab3d = cluster_labels.reshape(_B // _TCB, 1, _TCB)
    tc_center = pl.pallas_call(
        _tc_mse_kernel,
        grid=(_NB,),
        in_specs=[
            pl.BlockSpec((_TCB, _D), lambda i: (i + _SB // _TCB, 0)),
            pl.BlockSpec((1, 1, _TCB), lambda i: (i + _SB // _TCB, 0, 0)),
            pl.BlockSpec((_K, _D), lambda i: (0, 0)),
        ],
        out_specs=pl.BlockSpec((1, 1), lambda i: (0, 0)),
        out_shape=jax.ShapeDtypeStruct((1, 1), jnp.float32),
        scratch_shapes=[
            pltpu.VMEM((_K, _D), jnp.float32),
            pltpu.VMEM((_K, 1), jnp.float32),
            pltpu.SMEM((1,), jnp.float32),
        ],
    )(embeddings, lab3d, centers)
    sep = pl.pallas_call(
        _sep_kernel,
        out_shape=jax.ShapeDtypeStruct((1, 1), jnp.float32),
    )(centers)
    center_sum = jnp.sum(partials) + tc_center[0, 0]
    total = center_sum / (_B * _D) - _ALPHA * sep[0, 0] / (_K * (_K - 1))
    return total


# speedup vs baseline: 1.5918x; 1.0660x over previous
import functools

import jax
import jax.numpy as jnp
from jax import lax
from jax.experimental import pallas as pl
from jax.experimental.pallas import tpu as pltpu
from jax.experimental.pallas import tpu_sc as plsc

_B = 16384
_D = 128
_K = 100
_ALPHA = 0.5
_EPS = 1e-6

_SB = 4096
_TCB = 2048
_NB = (_B - _SB) // _TCB

_INFO = plsc.get_sparse_core_info()
_NC = _INFO.num_cores
_NS = _INFO.num_subcores
_NW = _NC * _NS
_RW = _SB // _NW
_CH = _RW
_NCHUNK = 1

_mesh = plsc.VectorSubcoreMesh(core_axis_name="c", subcore_axis_name="s")


@functools.partial(
    pl.kernel,
    mesh=_mesh,
    out_type=jax.ShapeDtypeStruct((_NW, 16), jnp.float32),
    compiler_params=pltpu.CompilerParams(needs_layout_passes=False),
    scratch_types=[
        pltpu.VMEM((_RW,), jnp.int32),
        pltpu.VMEM((_K, _D), jnp.float32),
        pltpu.VMEM_SHARED((_K, _D), jnp.float32),
        pltpu.VMEM((_CH, _D), jnp.float32),
        pltpu.VMEM((16,), jnp.float32),
        pltpu.SemaphoreType.DMA,
    ],
)
def _center_partials(emb_hbm, lab_hbm, cen_hbm, out_hbm,
                     lab_v, tab_v, tab_sp, emb0, acc_v, sem0):
    wid = lax.axis_index("s") * _NC + lax.axis_index("c")
    base = wid * _RW
    cp = pltpu.async_copy(emb_hbm.at[pl.ds(base, _CH)], emb0, sem0)

    @pl.when(lax.axis_index("s") == 0)
    def _():
        pltpu.sync_copy(cen_hbm, tab_sp)

    plsc.subcore_barrier()
    pltpu.sync_copy(tab_sp, tab_v)
    pltpu.sync_copy(lab_hbm.at[pl.ds(base, _RW)], lab_v)

    cols = [lax.iota(jnp.int32, 16) + g * 16 for g in range(_D // 16)]
    cp.wait()

    @plsc.parallel_loop(0, _CH, unroll=2, carry=jnp.zeros((16,), jnp.float32))
    def acc(r, a):
        lbl = plsc.load_gather(lab_v, [jnp.full((16,), r, jnp.int32)])
        for g in range(_D // 16):
            c = plsc.load_gather(tab_v, [lbl, cols[g]])
            e = emb0[r, pl.ds(g * 16, 16)]
            d = e - c
            a = a + d * d
        return a

    acc_v[...] = acc
    pltpu.sync_copy(acc_v, out_hbm.at[wid])


def _tc_kernel(emb_ref, lab_ref, cen_ref, out_ref, accm_ref, accc_ref, e2_ref):
    i = pl.program_id(0)

    @pl.when(i == 0)
    def _():
        accm_ref[...] = jnp.zeros_like(accm_ref)
        accc_ref[...] = jnp.zeros_like(accc_ref)
        e2_ref[0] = 0.0

    e = emb_ref[...]
    lab = lab_ref[0]
    kk = lax.broadcasted_iota(jnp.int32, (_K, _TCB), 0)
    onehot_t = jnp.where(lab == kk, 1.0, 0.0)
    m = lax.dot_general(onehot_t.astype(jnp.bfloat16), e.astype(jnp.bfloat16),
                        (((1,), (0,)), ((), ())),
                        preferred_element_type=jnp.float32)
    accm_ref[...] += m
    accc_ref[...] += jnp.sum(onehot_t, axis=1, keepdims=True)
    e2_ref[0] += jnp.sum(e * e)

    @pl.when(i == _NB - 1)
    def _():
        c = cen_ref[...]
        norms = jnp.sum(c * c, axis=1, keepdims=True)
        center_sum = (e2_ref[0]
                      - 2.0 * jnp.sum(accm_ref[...] * c)
                      + jnp.sum(accc_ref[...] * norms))
        gm = lax.dot_general(c, c, (((1,), (1,)), ((), ())),
                             precision=lax.Precision.HIGHEST)
        row = lax.broadcasted_iota(jnp.int32, (_K, _K), 0)
        col = lax.broadcasted_iota(jnp.int32, (_K, _K), 1)
        eye = jnp.where(row == col, 1.0, 0.0)
        n_col = jnp.sum(gm * eye, axis=1, keepdims=True)
        n_row = jnp.sum(gm * eye, axis=0, keepdims=True)
        s_col = jnp.sum(c, axis=1, keepdims=True)
        s_row = jnp.sum(eye * s_col, axis=0, keepdims=True)
        d2 = (n_col + n_row - 2.0 * gm
              + (2.0 * _EPS) * (s_col - s_row) + _D * _EPS * _EPS)
        dist = jnp.sqrt(jnp.maximum(d2, 0.0)) * (1.0 - eye)
        out_ref[...] = jnp.reshape(
            jnp.stack([center_sum, jnp.sum(dist)]), (1, 2))


def kernel(embeddings, cluster_labels, centers):
    partials = _center_partials(embeddings, cluster_labels, centers)
    lab3d = cluster_labels.reshape(_B // _TCB, 1, _TCB)
    tc_out = pl.pallas_call(
        _tc_kernel,
        grid=(_NB,),
        in_specs=[
            pl.BlockSpec((_TCB, _D), lambda i: (i + _SB // _TCB, 0)),
            pl.BlockSpec((1, 1, _TCB), lambda i: (i + _SB // _TCB, 0, 0)),
            pl.BlockSpec((_K, _D), lambda i: (0, 0)),
        ],
        out_specs=pl.BlockSpec((1, 2), lambda i: (0, 0)),
        out_shape=jax.ShapeDtypeStruct((1, 2), jnp.float32),
        scratch_shapes=[
            pltpu.VMEM((_K, _D), jnp.float32),
            pltpu.VMEM((_K, 1), jnp.float32),
            pltpu.SMEM((1,), jnp.float32),
        ],
    )(embeddings, lab3d, centers)
    center_sum = jnp.sum(partials) + tc_out[0, 0]
    total = center_sum / (_B * _D) - _ALPHA * tc_out[0, 1] / (_K * (_K - 1))
    return total


# speedup vs baseline: 1.7851x; 1.1214x over previous
import functools

import jax
import jax.numpy as jnp
from jax import lax
from jax.experimental import pallas as pl
from jax.experimental.pallas import tpu as pltpu
from jax.experimental.pallas import tpu_sc as plsc

_B = 16384
_D = 128
_K = 100
_ALPHA = 0.5
_EPS = 1e-6

_SB = 4096
_TCB = 4096
_NB = (_B - _SB) // _TCB

_INFO = plsc.get_sparse_core_info()
_NC = _INFO.num_cores
_NS = _INFO.num_subcores
_NW = _NC * _NS
_RW = _SB // _NW
_CH = _RW
_NCHUNK = 1

_mesh = plsc.VectorSubcoreMesh(core_axis_name="c", subcore_axis_name="s")


@functools.partial(
    pl.kernel,
    mesh=_mesh,
    out_type=jax.ShapeDtypeStruct((_NW, 16), jnp.float32),
    compiler_params=pltpu.CompilerParams(needs_layout_passes=False),
    scratch_types=[
        pltpu.VMEM((_RW,), jnp.int32),
        pltpu.VMEM((_K, _D), jnp.float32),
        pltpu.VMEM_SHARED((_K, _D), jnp.float32),
        pltpu.VMEM((_CH, _D), jnp.float32),
        pltpu.VMEM((16,), jnp.float32),
        pltpu.SemaphoreType.DMA,
    ],
)
def _center_partials(emb_hbm, lab_hbm, cen_hbm, out_hbm,
                     lab_v, tab_v, tab_sp, emb0, acc_v, sem0):
    wid = lax.axis_index("s") * _NC + lax.axis_index("c")
    base = wid * _RW
    cp = pltpu.async_copy(emb_hbm.at[pl.ds(base, _CH)], emb0, sem0)

    @pl.when(lax.axis_index("s") == 0)
    def _():
        pltpu.sync_copy(cen_hbm, tab_sp)

    plsc.subcore_barrier()
    pltpu.sync_copy(tab_sp, tab_v)
    pltpu.sync_copy(lab_hbm.at[pl.ds(base, _RW)], lab_v)

    cols = [lax.iota(jnp.int32, 16) + g * 16 for g in range(_D // 16)]
    cp.wait()

    @plsc.parallel_loop(0, _CH, unroll=2, carry=jnp.zeros((16,), jnp.float32))
    def acc(r, a):
        lbl = plsc.load_gather(lab_v, [jnp.full((16,), r, jnp.int32)])
        for g in range(_D // 16):
            c = plsc.load_gather(tab_v, [lbl, cols[g]])
            e = emb0[r, pl.ds(g * 16, 16)]
            d = e - c
            a = a + d * d
        return a

    acc_v[...] = acc
    pltpu.sync_copy(acc_v, out_hbm.at[wid])


def _tc_kernel(emb_ref, lab_ref, cen_ref, out_ref, accm_ref, accc_ref, e2_ref):
    i = pl.program_id(0)

    @pl.when(i == 0)
    def _():
        accm_ref[...] = jnp.zeros_like(accm_ref)
        accc_ref[...] = jnp.zeros_like(accc_ref)
        e2_ref[0] = 0.0

    e = emb_ref[...]
    lab = lab_ref[0]
    kk = lax.broadcasted_iota(jnp.int32, (_K, _TCB), 0)
    onehot_t = jnp.where(lab == kk, 1.0, 0.0)
    m = lax.dot_general(onehot_t.astype(jnp.bfloat16), e.astype(jnp.bfloat16),
                        (((1,), (0,)), ((), ())),
                        preferred_element_type=jnp.float32)
    accm_ref[...] += m
    accc_ref[...] += jnp.sum(onehot_t, axis=1, keepdims=True)
    e2_ref[0] += jnp.sum(e * e)

    @pl.when(i == _NB - 1)
    def _():
        c = cen_ref[...]
        norms = jnp.sum(c * c, axis=1, keepdims=True)
        center_sum = (e2_ref[0]
                      - 2.0 * jnp.sum(accm_ref[...] * c)
                      + jnp.sum(accc_ref[...] * norms))
        gm = lax.dot_general(c, c, (((1,), (1,)), ((), ())),
                             precision=lax.Precision.HIGHEST)
        row = lax.broadcasted_iota(jnp.int32, (_K, _K), 0)
        col = lax.broadcasted_iota(jnp.int32, (_K, _K), 1)
        eye = jnp.where(row == col, 1.0, 0.0)
        n_col = jnp.sum(gm * eye, axis=1, keepdims=True)
        n_row = jnp.sum(gm * eye, axis=0, keepdims=True)
        s_col = jnp.sum(c, axis=1, keepdims=True)
        s_row = jnp.sum(eye * s_col, axis=0, keepdims=True)
        d2 = (n_col + n_row - 2.0 * gm
              + (2.0 * _EPS) * (s_col - s_row) + _D * _EPS * _EPS)
        dist = jnp.sqrt(jnp.maximum(d2, 0.0)) * (1.0 - eye)
        out_ref[...] = jnp.reshape(
            jnp.stack([center_sum, jnp.sum(dist)]), (1, 2))


def _final_kernel(part_ref, tc_ref, out_ref):
    center_sum = jnp.sum(part_ref[...]) + tc_ref[0, 0]
    total = center_sum / (_B * _D) - _ALPHA * tc_ref[0, 1] / (_K * (_K - 1))
    out_ref[...] = jnp.reshape(total, (1, 1))


def kernel(embeddings, cluster_labels, centers):
    partials = _center_partials(embeddings, cluster_labels, centers)
    lab3d = cluster_labels.reshape(_B // _TCB, 1, _TCB)
    tc_out = pl.pallas_call(
        _tc_kernel,
        grid=(_NB,),
        in_specs=[
            pl.BlockSpec((_TCB, _D), lambda i: (i + _SB // _TCB, 0)),
            pl.BlockSpec((1, 1, _TCB), lambda i: (i + _SB // _TCB, 0, 0)),
            pl.BlockSpec((_K, _D), lambda i: (0, 0)),
        ],
        out_specs=pl.BlockSpec((1, 2), lambda i: (0, 0)),
        out_shape=jax.ShapeDtypeStruct((1, 2), jnp.float32),
        scratch_shapes=[
            pltpu.VMEM((_K, _D), jnp.float32),
            pltpu.VMEM((_K, 1), jnp.float32),
            pltpu.SMEM((1,), jnp.float32),
        ],
    )(embeddings, lab3d, centers)
    total = pl.pallas_call(
        _final_kernel,
        out_shape=jax.ShapeDtypeStruct((1, 1), jnp.float32),
    )(partials, tc_out)
    return total.reshape(())
